# baseline (speedup 1.0000x reference)
import functools

import jax
import jax.numpy as jnp
from jax import lax
from jax.experimental import pallas as pl
from jax.experimental.pallas import tpu as pltpu
from jax.experimental.pallas import tpu_sc as plsc

B, N, C, H, K, M, D = 4, 16384, 128, 8, 512, 32, 2
C_ = C // H
NCLUST = B * H * K
NU = NCLUST * M
ZROWS = 8192
BW = 128


def _maxpos_kernel(p_ref, o_ref):
    o_ref[...] = jnp.max(p_ref[...], axis=0, keepdims=True)


def _maxpos(pos2):
    return pl.pallas_call(
        _maxpos_kernel,
        out_shape=jax.ShapeDtypeStruct((1, D), jnp.float32),
    )(pos2)


NBLK = 1024


def _qkvt_kernel(x_ref, p_ref, w_ref, b_ref, wt_ref, mx_ref, o_ref):
    x = x_ref[0]
    y = jnp.dot(x, w_ref[...], preferred_element_type=jnp.float32) + b_ref[...]
    yb = y
    wtn = wt_ref[...] / mx_ref[...].reshape(D, 1)
    p = p_ref[0]
    t = p[:, 0:1] * wtn[0:1, :] + p[:, 1:2] * wtn[1:2, :]
    pad = jnp.zeros((NBLK, BW - 3 * C_ - 1), jnp.float32)
    rows = [
        jnp.concatenate(
            [yb[:, 48 * h:48 * h + 48], t[:, h:h + 1], pad], axis=1
        )[:, None, :]
        for h in range(H)
    ]
    o_ref[...] = jnp.concatenate(rows, axis=1).reshape(NBLK * H, BW)


def _qkvt(feat, pos, w, bvec, wt, maxpos):
    grid = (B, N // NBLK)
    return pl.pallas_call(
        _qkvt_kernel,
        grid=grid,
        in_specs=[
            pl.BlockSpec((1, NBLK, C), lambda b, i: (b, i, 0)),
            pl.BlockSpec((1, NBLK, D), lambda b, i: (b, i, 0)),
            pl.BlockSpec((C, 3 * C), lambda b, i: (0, 0)),
            pl.BlockSpec((1, 3 * C), lambda b, i: (0, 0)),
            pl.BlockSpec((D, H), lambda b, i: (0, 0)),
            pl.BlockSpec((1, D), lambda b, i: (0, 0)),
        ],
        out_specs=pl.BlockSpec(
            (NBLK * H, BW), lambda b, i: (b * (N // NBLK) + i, 0)
        ),
        out_shape=jax.ShapeDtypeStruct((B * N * H, BW), jnp.float32),
    )(feat, pos, w, bvec, wt, maxpos)


NW = 32
UPW = NU // NW
GW = 512
NWIN = UPW // GW


def _sc_gather(table, mi_flat):
    mesh = plsc.VectorSubcoreMesh(
        core_axis_name="c", subcore_axis_name="s", num_cores=2, num_subcores=16
    )

    @functools.partial(
        pl.kernel,
        mesh=mesh,
        out_type=jax.ShapeDtypeStruct((NU, BW), jnp.float32),
        scratch_types=[
            pltpu.VMEM((GW,), jnp.int32),
            pltpu.VMEM((GW,), jnp.int32),
            pltpu.VMEM((GW, BW), jnp.float32),
            pltpu.SemaphoreType.DMA,
        ],
    )
    def gather_k(tab, mi, go, mi_v, idx_v, buf, sem):
        wid = lax.axis_index("s") * 2 + lax.axis_index("c")
        b = wid >> 3
        h = wid & 7
        cst = b * (N * H) + h

        def win(j, _):
            base = wid * UPW + j * GW
            pltpu.sync_copy(mi.at[pl.ds(base, GW)], mi_v)

            def vec(i, _):
                mi16 = mi_v[pl.ds(i * 16, 16)]
                idx_v[pl.ds(i * 16, 16)] = mi16 * H + cst
                return 0

            lax.fori_loop(0, GW // 16, vec, 0, unroll=4)
            pltpu.async_copy(tab.at[idx_v], buf, sem).wait()
            pltpu.sync_copy(buf, go.at[pl.ds(base, GW)])
            return 0

        lax.fori_loop(0, NWIN, win, 0)

    return gather_k(table, mi_flat)


NCB = 128
P = 4
NEG = -1e30
ASTEPS = NU // (NCB * M)
AZSTEPS = ZROWS // (NCB * M)


def _attn_kernel(g_ref, o_ref):
    step = pl.program_id(0)
    g = g_ref[...].reshape(NCB // P, P * M, BW)
    q = g[:, :, 0:C_]
    kk = g[:, :, C_:2 * C_]
    v = g[:, :, 2 * C_:3 * C_]
    t = g[:, :, 3 * C_:3 * C_ + 1]
    ones = jnp.ones_like(t)
    qt = jnp.concatenate([q, ones], axis=2)
    kt = jnp.concatenate([kk, t], axis=2)
    s = lax.dot_general(qt, kt, (((2,), (2,)), ((0,), (0,))),
                        preferred_element_type=jnp.float32)
    ri = lax.broadcasted_iota(jnp.int32, (P * M, P * M), 0) // M
    ci = lax.broadcasted_iota(jnp.int32, (P * M, P * M), 1) // M
    allow = (ri == ci)[None, :, :]
    s = jnp.where(allow, s, NEG)
    mx = jnp.max(s, axis=2, keepdims=True)
    e = jnp.exp(s - mx)
    denom = jnp.sum(e, axis=2, keepdims=True)
    a = e / denom
    o = lax.dot_general(a, v, (((2,), (1,)), ((0,), (0,))),
                        preferred_element_type=jnp.float32)
    opad = jnp.concatenate(
        [o, jnp.zeros((NCB // P, P * M, BW - C_), jnp.float32)], axis=2
    ).reshape(NCB * M, BW)
    o_ref[...] = jnp.where(step < ASTEPS, opad, jnp.zeros_like(opad))


def _attention(g):
    grid = (ASTEPS + AZSTEPS,)
    return pl.pallas_call(
        _attn_kernel,
        grid=grid,
        in_specs=[
            pl.BlockSpec((NCB * M, BW), lambda i: (jnp.minimum(i, ASTEPS - 1), 0))
        ],
        out_specs=pl.BlockSpec((NCB * M, BW), lambda i: (i, 0)),
        out_shape=jax.ShapeDtypeStruct((NU + ZROWS, BW), jnp.float32),
    )(g)


def _proj_kernel(x_ref, w_ref, b_ref, o_ref):
    o_ref[...] = (
        jnp.dot(x_ref[...], w_ref[...], preferred_element_type=jnp.float32)
        + b_ref[...]
    )


def _proj(x, proj_w, proj_b):
    r, c = x.shape
    blk = 2048
    return pl.pallas_call(
        _proj_kernel,
        grid=(r // blk,),
        in_specs=[
            pl.BlockSpec((blk, c), lambda i: (i, 0)),
            pl.BlockSpec((c, c), lambda i: (0, 0)),
            pl.BlockSpec((1, c), lambda i: (0, 0)),
        ],
        out_specs=pl.BlockSpec((blk, c), lambda i: (i, 0)),
        out_shape=jax.ShapeDtypeStruct((r, c), jnp.float32),
    )(x, proj_w.T, proj_b[None, :])


def kernel(pos, feat, member_idx, cluster_mask, qkv_w, qkv_b, pos_mlp_w, pos_mlp_b, proj_w, proj_b):
    scale = C_ ** (-0.5)
    qpos = (jnp.arange(3 * C) // C_) % 3 == 0
    colscale = jnp.where(qpos, scale, 1.0).astype(jnp.float32)
    w = qkv_w.T * colscale[None, :]
    bvec = (qkv_b * colscale)[None, :]
    wt = pos_mlp_w[:, :, 0].T

    maxpos = _maxpos(pos.reshape(B * N, D))
    table = _qkvt(feat, pos, w, bvec, wt, maxpos)

    g = _sc_gather(table, member_idx.reshape(-1).astype(jnp.int32))

    feato = _attention(g)

    featu = lax.optimization_barrier(feato[:NU, :C_])
    batch_idx = jnp.repeat(jnp.arange(B * H), K * M)
    mi = member_idx.reshape(-1)
    new_feat = jnp.zeros((B * H, N, C_), jnp.float32).at[batch_idx, mi].set(featu)
    feat2 = jnp.transpose(new_feat.reshape(B, H, N, C_), (0, 2, 1, 3)).reshape(B * N, C)

    out = _proj(feat2, proj_w, proj_b)
    return out.reshape(B, N, C)


# speedup vs baseline: 2.4226x; 2.4226x over previous
import functools

import jax
import jax.numpy as jnp
from jax import lax
from jax.experimental import pallas as pl
from jax.experimental.pallas import tpu as pltpu
from jax.experimental.pallas import tpu_sc as plsc

B, N, C, H, K, M, D = 4, 16384, 128, 8, 512, 32, 2
C_ = C // H
NCLUST = B * H * K
NU = NCLUST * M
ZROWS = 8192
BW = 128


def _maxpos_kernel(p_ref, o_ref):
    o_ref[...] = jnp.max(p_ref[...], axis=0, keepdims=True)


def _maxpos(pos2):
    return pl.pallas_call(
        _maxpos_kernel,
        out_shape=jax.ShapeDtypeStruct((1, D), jnp.float32),
    )(pos2)


NBLK = 1024


def _qkvt_kernel(x_ref, p_ref, w_ref, b_ref, wt_ref, mx_ref, o_ref):
    x = x_ref[0]
    y = jnp.dot(x, w_ref[...], preferred_element_type=jnp.float32) + b_ref[...]
    yb = y
    wtn = wt_ref[...] / mx_ref[...].reshape(D, 1)
    p = p_ref[0]
    t = p[:, 0:1] * wtn[0:1, :] + p[:, 1:2] * wtn[1:2, :]
    pad = jnp.zeros((NBLK, BW - 3 * C_ - 1), jnp.float32)
    rows = [
        jnp.concatenate(
            [yb[:, 48 * h:48 * h + 48], t[:, h:h + 1], pad], axis=1
        )[:, None, :]
        for h in range(H)
    ]
    o_ref[...] = jnp.concatenate(rows, axis=1).reshape(NBLK * H, BW)


def _qkvt(feat, pos, w, bvec, wt, maxpos):
    grid = (B, N // NBLK)
    return pl.pallas_call(
        _qkvt_kernel,
        grid=grid,
        in_specs=[
            pl.BlockSpec((1, NBLK, C), lambda b, i: (b, i, 0)),
            pl.BlockSpec((1, NBLK, D), lambda b, i: (b, i, 0)),
            pl.BlockSpec((C, 3 * C), lambda b, i: (0, 0)),
            pl.BlockSpec((1, 3 * C), lambda b, i: (0, 0)),
            pl.BlockSpec((D, H), lambda b, i: (0, 0)),
            pl.BlockSpec((1, D), lambda b, i: (0, 0)),
        ],
        out_specs=pl.BlockSpec(
            (NBLK * H, BW), lambda b, i: (b * (N // NBLK) + i, 0)
        ),
        out_shape=jax.ShapeDtypeStruct((B * N * H, BW), jnp.float32),
    )(feat, pos, w, bvec, wt, maxpos)


NW = 32
UPW = NU // NW
GW = 512
NWIN = UPW // GW


def _sc_gather(table, mi_flat):
    mesh = plsc.VectorSubcoreMesh(
        core_axis_name="c", subcore_axis_name="s", num_cores=2, num_subcores=16
    )

    @functools.partial(
        pl.kernel,
        mesh=mesh,
        out_type=jax.ShapeDtypeStruct((NU, BW), jnp.float32),
        scratch_types=[
            pltpu.VMEM((GW,), jnp.int32),
            pltpu.VMEM((GW,), jnp.int32),
            pltpu.VMEM((GW, BW), jnp.float32),
            pltpu.SemaphoreType.DMA,
        ],
    )
    def gather_k(tab, mi, go, mi_v, idx_v, buf, sem):
        wid = lax.axis_index("s") * 2 + lax.axis_index("c")
        b = wid >> 3
        h = wid & 7
        cst = b * (N * H) + h

        def win(j, _):
            base = wid * UPW + j * GW
            pltpu.sync_copy(mi.at[pl.ds(base, GW)], mi_v)

            def vec(i, _):
                mi16 = mi_v[pl.ds(i * 16, 16)]
                idx_v[pl.ds(i * 16, 16)] = mi16 * H + cst
                return 0

            lax.fori_loop(0, GW // 16, vec, 0, unroll=4)
            pltpu.async_copy(tab.at[idx_v], buf, sem).wait()
            pltpu.sync_copy(buf, go.at[pl.ds(base, GW)])
            return 0

        lax.fori_loop(0, NWIN, win, 0)

    return gather_k(table, mi_flat)


NCB = 128
P = 4
NEG = -1e30
ASTEPS = NU // (NCB * M)
AZSTEPS = ZROWS // (NCB * M)


def _attn_kernel(g_ref, o_ref):
    step = pl.program_id(0)
    g = g_ref[...].reshape(NCB // P, P * M, BW)
    q = g[:, :, 0:C_]
    kk = g[:, :, C_:2 * C_]
    v = g[:, :, 2 * C_:3 * C_]
    t = g[:, :, 3 * C_:3 * C_ + 1]
    ones = jnp.ones_like(t)
    qt = jnp.concatenate([q, ones], axis=2)
    kt = jnp.concatenate([kk, t], axis=2)
    s = lax.dot_general(qt, kt, (((2,), (2,)), ((0,), (0,))),
                        preferred_element_type=jnp.float32)
    ri = lax.broadcasted_iota(jnp.int32, (P * M, P * M), 0) // M
    ci = lax.broadcasted_iota(jnp.int32, (P * M, P * M), 1) // M
    allow = (ri == ci)[None, :, :]
    s = jnp.where(allow, s, NEG)
    mx = jnp.max(s, axis=2, keepdims=True)
    e = jnp.exp(s - mx)
    denom = jnp.sum(e, axis=2, keepdims=True)
    a = e / denom
    o = lax.dot_general(a, v, (((2,), (1,)), ((0,), (0,))),
                        preferred_element_type=jnp.float32)
    opad = jnp.concatenate(
        [o, jnp.zeros((NCB // P, P * M, BW - C_), jnp.float32)], axis=2
    ).reshape(NCB * M, BW)
    o_ref[...] = jnp.where(step < ASTEPS, opad, jnp.zeros_like(opad))


def _attention(g):
    grid = (ASTEPS + AZSTEPS,)
    return pl.pallas_call(
        _attn_kernel,
        grid=grid,
        in_specs=[
            pl.BlockSpec((NCB * M, BW), lambda i: (jnp.minimum(i, ASTEPS - 1), 0))
        ],
        out_specs=pl.BlockSpec((NCB * M, BW), lambda i: (i, 0)),
        out_shape=jax.ShapeDtypeStruct((NU + ZROWS, BW), jnp.float32),
    )(g)


def _proj_kernel(x_ref, w_ref, b_ref, o_ref):
    o_ref[...] = (
        jnp.dot(x_ref[...], w_ref[...], preferred_element_type=jnp.float32)
        + b_ref[...]
    )


def _proj(x, proj_w, proj_b):
    r, c = x.shape
    blk = 2048
    return pl.pallas_call(
        _proj_kernel,
        grid=(r // blk,),
        in_specs=[
            pl.BlockSpec((blk, c), lambda i: (i, 0)),
            pl.BlockSpec((c, c), lambda i: (0, 0)),
            pl.BlockSpec((1, c), lambda i: (0, 0)),
        ],
        out_specs=pl.BlockSpec((blk, c), lambda i: (i, 0)),
        out_shape=jax.ShapeDtypeStruct((r, c), jnp.float32),
    )(x, proj_w.T, proj_b[None, :])


def kernel(pos, feat, member_idx, cluster_mask, qkv_w, qkv_b, pos_mlp_w, pos_mlp_b, proj_w, proj_b):
    scale = C_ ** (-0.5)
    qpos = (jnp.arange(3 * C) // C_) % 3 == 0
    colscale = jnp.where(qpos, scale, 1.0).astype(jnp.float32)
    w = qkv_w.T * colscale[None, :]
    bvec = (qkv_b * colscale)[None, :]
    wt = pos_mlp_w[:, :, 0].T

    maxpos = _maxpos(pos.reshape(B * N, D))
    table = _qkvt(feat, pos, w, bvec, wt, maxpos)

    g = _sc_gather(table, member_idx.reshape(-1).astype(jnp.int32))

    feato = _attention(g)

    featc = lax.optimization_barrier(feato[:, :C_])
    mi = member_idx.reshape(-1)
    bh = jnp.repeat(jnp.arange(B * H, dtype=jnp.int32), K * M)
    flat_tgt = bh * N + mi.astype(jnp.int32)
    init = -1 - (jnp.arange(B * H * N, dtype=jnp.int32) & (ZROWS - 1))
    u = jnp.arange(NU, dtype=jnp.int32)
    w = init.at[flat_tgt].max(u)
    winner = jnp.where(w >= 0, w, NU - 1 - w)
    gathered = jnp.take(featc, winner, axis=0)
    feat2 = jnp.transpose(
        gathered.reshape(B, H, N, C_), (0, 2, 1, 3)
    ).reshape(B * N, C)

    out = _proj(feat2, proj_w, proj_b)
    return out.reshape(B, N, C)
